# Initial kernel scaffold; baseline (speedup 1.0000x reference)
import jax
import jax.numpy as jnp
from jax.experimental import pallas as pl


def kernel(x, embedding):
    raise NotImplementedError("write your pallas kernel here")


# speedup vs baseline: 1.0526x; 1.0526x over previous
import functools

import jax
import jax.numpy as jnp
from jax import lax
from jax.experimental import pallas as pl
from jax.experimental.pallas import tpu as pltpu

NUM_EMB = 8192
DIM = 256
N_TOK = 4096
COMMIT = 0.25

BN = 256
BM = 2048
BN_C = 128


def _argmin_body(xsq_ref, esq_ref, x_ref, e_ref, codes_ref, loss_ref,
                 mn_ref, ai_ref, acc_ref):
    n = pl.program_id(0)
    m = pl.program_id(1)
    n_m = pl.num_programs(1)

    @pl.when(m == 0)
    def _init():
        mn_ref[...] = jnp.full((BN, 1), jnp.inf, jnp.float32)
        ai_ref[...] = jnp.zeros((BN, 1), jnp.int32)

    mm = lax.dot_general(x_ref[...], e_ref[...],
                         (((1,), (1,)), ((), ())),
                         preferred_element_type=jnp.float32)
    d = (xsq_ref[...] + esq_ref[...]) - 2.0 * mm
    bmin = jnp.min(d, axis=1, keepdims=True)
    col = lax.broadcasted_iota(jnp.int32, (BN, BM), 1)
    barg = jnp.min(jnp.where(d == bmin, col, BM), axis=1, keepdims=True)
    barg = barg + m * BM
    better = bmin < mn_ref[...]
    ai_ref[...] = jnp.where(better, barg, ai_ref[...])
    mn_ref[...] = jnp.where(better, bmin, mn_ref[...])

    @pl.when(m == n_m - 1)
    def _fin():
        codes_ref[...] = ai_ref[...].reshape(1, 1, BN)
        blk_loss = jnp.sum(mn_ref[...], axis=0, keepdims=True)
        prev = acc_ref[...]
        new_acc = jnp.where(n == 0, jnp.zeros_like(prev), prev) + blk_loss
        acc_ref[...] = new_acc
        loss_ref[...] = new_acc * (COMMIT / (N_TOK * DIM))


def _onehot_body(codes_ref, emb_ref, oh_ref, q_ref, perp_ref, cnt_ref):
    i = pl.program_id(0)
    n_i = pl.num_programs(0)
    c = codes_ref[...].reshape(BN_C, 1)
    col = lax.broadcasted_iota(jnp.int32, (BN_C, NUM_EMB), 1)
    oh = (col == c).astype(jnp.float32)
    oh_ref[...] = oh
    q_ref[...] = lax.dot_general(oh, emb_ref[...],
                                 (((1,), (0,)), ((), ())),
                                 preferred_element_type=jnp.float32)
    new_cnt = jnp.where(i == 0, jnp.zeros_like(cnt_ref[...]),
                        cnt_ref[...]) + jnp.sum(oh, axis=0, keepdims=True)
    cnt_ref[...] = new_cnt

    @pl.when(i == n_i - 1)
    def _fin():
        p = new_cnt * (1.0 / N_TOK)
        ent = jnp.sum(p * jnp.log(p + 1e-10), axis=1, keepdims=True)
        perp_ref[...] = jnp.exp(-ent)


def kernel(x, embedding):
    x_flat = x.reshape(-1, DIM)
    e_sq = jnp.sum(embedding ** 2, axis=1)
    x_sq = jnp.sum(x_flat ** 2, axis=1, keepdims=True)

    n_blocks = N_TOK // BN
    codes3, loss2 = pl.pallas_call(
        _argmin_body,
        grid=(n_blocks, NUM_EMB // BM),
        in_specs=[
            pl.BlockSpec((BN, 1), lambda n, m: (n, 0)),
            pl.BlockSpec((1, BM), lambda n, m: (0, m)),
            pl.BlockSpec((BN, DIM), lambda n, m: (n, 0)),
            pl.BlockSpec((BM, DIM), lambda n, m: (m, 0)),
        ],
        out_specs=[
            pl.BlockSpec((1, 1, BN), lambda n, m: (n, 0, 0)),
            pl.BlockSpec((1, 1), lambda n, m: (0, 0)),
        ],
        out_shape=[
            jax.ShapeDtypeStruct((n_blocks, 1, BN), jnp.int32),
            jax.ShapeDtypeStruct((1, 1), jnp.float32),
        ],
        scratch_shapes=[
            pltpu.VMEM((BN, 1), jnp.float32),
            pltpu.VMEM((BN, 1), jnp.int32),
            pltpu.VMEM((1, 1), jnp.float32),
        ],
        compiler_params=pltpu.CompilerParams(
            dimension_semantics=("arbitrary", "arbitrary")),
    )(x_sq, e_sq.reshape(1, NUM_EMB), x_flat, embedding)

    nc_blocks = N_TOK // BN_C
    one_hot2, q2, perp2 = pl.pallas_call(
        _onehot_body,
        grid=(nc_blocks,),
        in_specs=[
            pl.BlockSpec((1, 1, BN_C),
                         lambda i: (i, 0, 0)),
            pl.BlockSpec((NUM_EMB, DIM), lambda i: (0, 0)),
        ],
        out_specs=[
            pl.BlockSpec((BN_C, NUM_EMB), lambda i: (i, 0)),
            pl.BlockSpec((BN_C, DIM), lambda i: (i, 0)),
            pl.BlockSpec((1, 1), lambda i: (0, 0)),
        ],
        out_shape=[
            jax.ShapeDtypeStruct((N_TOK, NUM_EMB), jnp.float32),
            jax.ShapeDtypeStruct((N_TOK, DIM), jnp.float32),
            jax.ShapeDtypeStruct((1, 1), jnp.float32),
        ],
        scratch_shapes=[
            pltpu.VMEM((1, NUM_EMB), jnp.float32),
        ],
        compiler_params=pltpu.CompilerParams(
            dimension_semantics=("arbitrary",)),
    )(codes3.reshape(nc_blocks, 1, BN_C), embedding)

    B, T, _ = x.shape
    codes = codes3.reshape(B, T)
    quantized_st = q2.reshape(x.shape)
    one_hot = one_hot2.reshape(B, T, NUM_EMB)
    loss = loss2[0, 0]
    perplexity = perp2[0, 0]
    return quantized_st, codes, one_hot, loss, perplexity
